# baseline (speedup 1.0000x reference)
import functools

import jax
import jax.numpy as jnp
from jax import lax
from jax.experimental import pallas as pl
from jax.experimental.pallas import tpu as pltpu
from jax.experimental.pallas import tpu_sc as plsc

V = 1000
VP = 1024
N = 1024 * 50
NH = N // 2
NC, NS, L = 2, 16, 16
NW = NC * NS
BPW = NH // NW
CHUNK = 16
NCHUNK = BPW // CHUNK


def _lz_body(t_ref, o_ref):
    t = t_ref[...]
    m = jnp.max(t, axis=1, keepdims=True)
    s = jnp.sum(jnp.exp(t - m), axis=1, keepdims=True)
    o_ref[...] = jnp.log(s) + m


def _fin_body(p_ref, o_ref):
    o_ref[...] = jnp.sum(p_ref[...], axis=(0, 1), keepdims=True) * (1.0 / N)


def _sc_body(table_h, toks_h, tgts_h, lz_h, out_h, part_h,
             idx_v, tgt_v, lz_v, rows0_v, rows1_v, acc_v,
             gsem0, gsem1, ssem0, ssem1):
    c = lax.axis_index("c")
    s = lax.axis_index("s")
    wid = s * NC + c
    pltpu.sync_copy(toks_h.at[wid], idx_v)
    pltpu.sync_copy(tgts_h.at[wid], tgt_v)
    pltpu.sync_copy(lz_h, lz_v)
    acc_v[...] = jnp.zeros((L,), jnp.float32)
    base = wid * BPW

    rows = (rows0_v, rows1_v)
    gsem = (gsem0, gsem1)
    ssem = (ssem0, ssem1)

    def gather_start(g, b):
        pltpu.async_copy(table_h.at[idx_v.at[pl.ds(g * CHUNK, CHUNK)]],
                         rows[b], gsem[b])

    def gather_wait(b):
        pltpu.make_async_copy(table_h.at[pl.ds(0, CHUNK)], rows[b],
                              gsem[b]).wait()

    def scatter_start(g, b):
        pltpu.async_copy(rows[b], out_h.at[pl.ds(base + g * CHUNK, CHUNK)],
                         ssem[b])

    def scatter_wait(b):
        pltpu.make_async_copy(rows[b], out_h.at[pl.ds(0, CHUNK)],
                              ssem[b]).wait()

    def loss_partial(g, b):
        for k in range(CHUNK // L):
            rowid = lax.iota(jnp.int32, L) + (k * L)
            tg = tgt_v[pl.ds(g * CHUNK + k * L, L)]
            tk = idx_v[pl.ds(g * CHUNK + k * L, L)]
            val = plsc.load_gather(rows[b], [rowid, tg])
            lzv = plsc.load_gather(lz_v, [tk])
            acc_v[...] = acc_v[...] + (lzv - val)

    gather_start(0, 0)
    gather_wait(0)
    scatter_start(0, 0)
    gather_start(1, 1)
    loss_partial(0, 0)

    def steady(t, carry):
        for j in range(2):
            g = 2 * t + 1 + j
            b = 1 - j
            gather_wait(b)
            scatter_start(g, b)
            scatter_wait(1 - b)
            gather_start(g + 1, 1 - b)
            loss_partial(g, b)
        return carry

    lax.fori_loop(0, (NCHUNK - 2) // 2, steady, 0)

    g = NCHUNK - 1
    gather_wait(1)
    scatter_start(g, 1)
    scatter_wait(0)
    loss_partial(g, 1)
    scatter_wait(1)
    pltpu.sync_copy(acc_v, part_h.at[wid])


def kernel(tokens, target, embedding_weight):
    table = embedding_weight.astype(jnp.float32)
    toks_f = tokens.reshape(-1).astype(jnp.int32)
    tgts_f = target.reshape(-1).astype(jnp.int32)

    lz2 = pl.pallas_call(
        _lz_body,
        out_shape=jax.ShapeDtypeStruct((V, 1), jnp.float32),
    )(table)
    lz = lz2.reshape(V)

    mesh = plsc.VectorSubcoreMesh(core_axis_name="c", subcore_axis_name="s")
    sc = functools.partial(
        pl.kernel,
        mesh=mesh,
        compiler_params=pltpu.CompilerParams(
            use_tc_tiling_on_sc=True, needs_layout_passes=False),
        out_type=[
            jax.ShapeDtypeStruct((NH, VP), jnp.float32),
            jax.ShapeDtypeStruct((NW, L), jnp.float32),
        ],
        scratch_types=[
            pltpu.VMEM((BPW,), jnp.int32),
            pltpu.VMEM((BPW,), jnp.int32),
            pltpu.VMEM((V,), jnp.float32),
            pltpu.VMEM((CHUNK, VP), jnp.float32),
            pltpu.VMEM((CHUNK, VP), jnp.float32),
            pltpu.VMEM((L,), jnp.float32),
            pltpu.SemaphoreType.DMA,
            pltpu.SemaphoreType.DMA,
            pltpu.SemaphoreType.DMA,
            pltpu.SemaphoreType.DMA,
        ],
    )(_sc_body)
    table_p = jnp.pad(table, ((0, 0), (0, VP - V)))
    out_a, parts_a = sc(table_p, toks_f[:NH].reshape(NW, BPW),
                        tgts_f[:NH].reshape(NW, BPW), lz)
    out_b, parts_b = sc(table_p, toks_f[NH:].reshape(NW, BPW),
                        tgts_f[NH:].reshape(NW, BPW), lz)

    logits = jnp.concatenate([out_a[:, :V], out_b[:, :V]], axis=0)

    loss2 = pl.pallas_call(
        _fin_body,
        out_shape=jax.ShapeDtypeStruct((1, 1), jnp.float32),
    )(jnp.concatenate([parts_a, parts_b], axis=0))
    loss = loss2[0, 0]
    return (logits, loss)


# speedup vs baseline: 1.4890x; 1.4890x over previous
import functools

import jax
import jax.numpy as jnp
from jax import lax
from jax.experimental import pallas as pl
from jax.experimental.pallas import tpu as pltpu
from jax.experimental.pallas import tpu_sc as plsc

V = 1000
VP = 1024
N = 1024 * 50
NC, NS, L = 2, 16, 16
NW = NC * NS
BPW = N // NW
CHUNK = 32
NCHUNK = BPW // CHUNK


def _lz_body(t_ref, o_ref):
    t = t_ref[...]
    m = jnp.max(t, axis=1, keepdims=True)
    s = jnp.sum(jnp.exp(t - m), axis=1, keepdims=True)
    o_ref[...] = jnp.log(s) + m


def _fin_body(p_ref, o_ref):
    o_ref[...] = jnp.sum(p_ref[...], axis=(0, 1), keepdims=True) * (1.0 / N)


def _sc_body(table_h, toks_h, tgts_h, lz_h, out_h, part_h,
             idx_v, tgt_v, lz_v, rows0_v, rows1_v, acc_v,
             gsem0, gsem1, ssem0, ssem1):
    c = lax.axis_index("c")
    s = lax.axis_index("s")
    wid = s * NC + c
    pltpu.sync_copy(toks_h.at[wid], idx_v)
    pltpu.sync_copy(tgts_h.at[wid], tgt_v)
    pltpu.sync_copy(lz_h, lz_v)
    acc_v[...] = jnp.zeros((L,), jnp.float32)
    base = wid * BPW

    rows = (rows0_v, rows1_v)
    gsem = (gsem0, gsem1)
    ssem = (ssem0, ssem1)

    def gather_start(g, b):
        pltpu.async_copy(table_h.at[idx_v.at[pl.ds(g * CHUNK, CHUNK)]],
                         rows[b], gsem[b])

    def gather_wait(b):
        pltpu.make_async_copy(table_h.at[pl.ds(0, CHUNK)], rows[b],
                              gsem[b]).wait()

    def scatter_start(g, b):
        pltpu.async_copy(rows[b], out_h.at[pl.ds(base + g * CHUNK, CHUNK)],
                         ssem[b])

    def scatter_wait(b):
        pltpu.make_async_copy(rows[b], out_h.at[pl.ds(0, CHUNK)],
                              ssem[b]).wait()

    def loss_partial(g, b):
        for k in range(CHUNK // L):
            rowid = lax.iota(jnp.int32, L) + (k * L)
            tg = tgt_v[pl.ds(g * CHUNK + k * L, L)]
            tk = idx_v[pl.ds(g * CHUNK + k * L, L)]
            val = plsc.load_gather(rows[b], [rowid, tg])
            lzv = plsc.load_gather(lz_v, [tk])
            acc_v[...] = acc_v[...] + (lzv - val)

    gather_start(0, 0)
    gather_wait(0)
    scatter_start(0, 0)
    gather_start(1, 1)
    loss_partial(0, 0)

    def steady(t, carry):
        for j in range(2):
            g = 2 * t + 1 + j
            b = 1 - j
            gather_wait(b)
            scatter_start(g, b)
            scatter_wait(1 - b)
            gather_start(g + 1, 1 - b)
            loss_partial(g, b)
        return carry

    lax.fori_loop(0, (NCHUNK - 2) // 2, steady, 0)

    g = NCHUNK - 1
    gather_wait(1)
    scatter_start(g, 1)
    scatter_wait(0)
    loss_partial(g, 1)
    scatter_wait(1)
    pltpu.sync_copy(acc_v, part_h.at[wid])


def kernel(tokens, target, embedding_weight):
    table = embedding_weight.astype(jnp.float32)
    toks_f = tokens.reshape(-1).astype(jnp.int32)
    tgts_f = target.reshape(-1).astype(jnp.int32)

    lz2 = pl.pallas_call(
        _lz_body,
        out_shape=jax.ShapeDtypeStruct((V, 1), jnp.float32),
    )(table)
    lz = lz2.reshape(V)

    mesh = plsc.VectorSubcoreMesh(core_axis_name="c", subcore_axis_name="s")
    sc = functools.partial(
        pl.kernel,
        mesh=mesh,
        compiler_params=pltpu.CompilerParams(
            use_tc_tiling_on_sc=True, needs_layout_passes=False),
        out_type=[
            jax.ShapeDtypeStruct((N, VP), jnp.float32),
            jax.ShapeDtypeStruct((NW, L), jnp.float32),
        ],
        scratch_types=[
            pltpu.VMEM((BPW,), jnp.int32),
            pltpu.VMEM((BPW,), jnp.int32),
            pltpu.VMEM((V,), jnp.float32),
            pltpu.VMEM((CHUNK, VP), jnp.float32),
            pltpu.VMEM((CHUNK, VP), jnp.float32),
            pltpu.VMEM((L,), jnp.float32),
            pltpu.SemaphoreType.DMA,
            pltpu.SemaphoreType.DMA,
            pltpu.SemaphoreType.DMA,
            pltpu.SemaphoreType.DMA,
        ],
    )(_sc_body)
    table_p = jnp.pad(table, ((0, 0), (0, VP - V)))
    logits_p, parts = sc(table_p, toks_f.reshape(NW, BPW),
                         tgts_f.reshape(NW, BPW), lz)
    logits = logits_p[:, :V]

    loss2 = pl.pallas_call(
        _fin_body,
        out_shape=jax.ShapeDtypeStruct((1, 1), jnp.float32),
    )(parts)
    loss = loss2[0, 0]
    return (logits, loss)
